# baseline (speedup 1.0000x reference)
import functools

import jax
import jax.numpy as jnp
from jax.experimental import pallas as pl
from jax.experimental.pallas import tpu as pltpu

_TILE = 4000


def _gcn_body(adjt_ref, x_ref, w_ref, b_ref, o_ref, acc_ref):
    i = pl.program_id(0)
    nt = pl.num_programs(0)

    @pl.when(i == 0)
    def _init():
        acc_ref[...] = jnp.zeros_like(acc_ref)

    acc_ref[...] += jax.lax.dot_general(
        adjt_ref[...],
        x_ref[...],
        dimension_numbers=(((0,), (0,)), ((), ())),
        preferred_element_type=jnp.float32,
    )

    @pl.when(i == nt - 1)
    def _finish():
        o_ref[...] = (
            jnp.dot(acc_ref[...], w_ref[...], preferred_element_type=jnp.float32)
            + b_ref[...]
        )


@jax.jit
def kernel(x, adj, weight, bias):
    n, f_in = x.shape
    f_out = adj.shape[0]
    tile = _TILE if n % _TILE == 0 else n
    nt = n // tile
    adjt = jnp.swapaxes(adj, 0, 1)
    bias2 = bias.reshape(1, f_out)
    return pl.pallas_call(
        _gcn_body,
        grid=(nt,),
        in_specs=[
            pl.BlockSpec((tile, f_out), lambda i: (i, 0)),
            pl.BlockSpec((tile, f_in), lambda i: (i, 0)),
            pl.BlockSpec((f_in, f_out), lambda i: (0, 0)),
            pl.BlockSpec((1, f_out), lambda i: (0, 0)),
        ],
        out_specs=pl.BlockSpec((f_out, f_out), lambda i: (0, 0)),
        out_shape=jax.ShapeDtypeStruct((f_out, f_out), jnp.float32),
        scratch_shapes=[pltpu.VMEM((f_out, f_out), jnp.float32)],
        compiler_params=pltpu.CompilerParams(
            dimension_semantics=("arbitrary",),
        ),
    )(adjt, x, weight, bias2)


# speedup vs baseline: 1.1250x; 1.1250x over previous
import functools

import jax
import jax.numpy as jnp
from jax.experimental import pallas as pl
from jax.experimental.pallas import tpu as pltpu

_TILE = 10000


def _gcn_body(adjt_ref, x_ref, w_ref, b_ref, o_ref, acc_ref):
    i = pl.program_id(0)
    nt = pl.num_programs(0)

    @pl.when(i == 0)
    def _init():
        acc_ref[...] = jnp.zeros_like(acc_ref)

    acc_ref[...] += jax.lax.dot_general(
        adjt_ref[...].astype(jnp.bfloat16),
        x_ref[...].astype(jnp.bfloat16),
        dimension_numbers=(((0,), (0,)), ((), ())),
        preferred_element_type=jnp.float32,
    )

    @pl.when(i == nt - 1)
    def _finish():
        o_ref[...] = (
            jnp.dot(acc_ref[...], w_ref[...], preferred_element_type=jnp.float32)
            + b_ref[...]
        )


@jax.jit
def kernel(x, adj, weight, bias):
    n, f_in = x.shape
    f_out = adj.shape[0]
    tile = _TILE if n % _TILE == 0 else n
    nt = n // tile
    adjt = jnp.swapaxes(adj, 0, 1)
    bias2 = bias.reshape(1, f_out)
    return pl.pallas_call(
        _gcn_body,
        grid=(nt,),
        in_specs=[
            pl.BlockSpec((tile, f_out), lambda i: (i, 0)),
            pl.BlockSpec((tile, f_in), lambda i: (i, 0)),
            pl.BlockSpec((f_in, f_out), lambda i: (0, 0)),
            pl.BlockSpec((1, f_out), lambda i: (0, 0)),
        ],
        out_specs=pl.BlockSpec((f_out, f_out), lambda i: (0, 0)),
        out_shape=jax.ShapeDtypeStruct((f_out, f_out), jnp.float32),
        scratch_shapes=[pltpu.VMEM((f_out, f_out), jnp.float32)],
        compiler_params=pltpu.CompilerParams(
            dimension_semantics=("arbitrary",),
        ),
    )(adjt, x, weight, bias2)


# speedup vs baseline: 1.1356x; 1.0094x over previous
import functools

import jax
import jax.numpy as jnp
from jax.experimental import pallas as pl
from jax.experimental.pallas import tpu as pltpu

_TILE = 10000


def _gcn_body(adjt_ref, x_ref, w_ref, b_ref, o_ref, acc_ref):
    i = pl.program_id(0)
    nt = pl.num_programs(0)

    @pl.when(i == 0)
    def _init():
        acc_ref[...] = jnp.zeros_like(acc_ref)

    acc_ref[...] += jax.lax.dot_general(
        adjt_ref[...],
        x_ref[...],
        dimension_numbers=(((0,), (0,)), ((), ())),
        preferred_element_type=jnp.float32,
    )

    @pl.when(i == nt - 1)
    def _finish():
        o_ref[...] = (
            jnp.dot(acc_ref[...], w_ref[...], preferred_element_type=jnp.float32)
            + b_ref[...]
        )


@jax.jit
def kernel(x, adj, weight, bias):
    n, f_in = x.shape
    f_out = adj.shape[0]
    tile = _TILE if n % _TILE == 0 else n
    nt = n // tile
    adjt = jnp.swapaxes(adj, 0, 1)
    bias2 = bias.reshape(1, f_out)
    return pl.pallas_call(
        _gcn_body,
        grid=(nt,),
        in_specs=[
            pl.BlockSpec((tile, f_out), lambda i: (i, 0)),
            pl.BlockSpec((tile, f_in), lambda i: (i, 0)),
            pl.BlockSpec((f_in, f_out), lambda i: (0, 0)),
            pl.BlockSpec((1, f_out), lambda i: (0, 0)),
        ],
        out_specs=pl.BlockSpec((f_out, f_out), lambda i: (0, 0)),
        out_shape=jax.ShapeDtypeStruct((f_out, f_out), jnp.float32),
        scratch_shapes=[pltpu.VMEM((f_out, f_out), jnp.float32)],
        compiler_params=pltpu.CompilerParams(
            dimension_semantics=("arbitrary",),
        ),
    )(adjt, x, weight, bias2)
